# Initial kernel scaffold; baseline (speedup 1.0000x reference)
import jax
import jax.numpy as jnp
from jax.experimental import pallas as pl


def kernel(step, lprobs, scores):
    raise NotImplementedError("write your pallas kernel here")


# speedup vs baseline: 1.4310x; 1.4310x over previous
import functools

import jax
import jax.numpy as jnp
from jax.experimental import pallas as pl

_G = 4
_ALPHA = 0.5
_K = 4
_NEG = -3.0e38


def _tc_body(lprobs_ref, bias_ref, outs_ref, outi_ref, outb_ref):
    V = lprobs_ref.shape[-1]
    iota = jax.lax.broadcasted_iota(jnp.int32, (1, V), 1)
    pen = jnp.zeros((1, V), jnp.float32)
    BIG = jnp.int32(2**30)
    oiota = jax.lax.broadcasted_iota(jnp.int32, (1, 1, _G * _K), 2)
    svec = jnp.zeros((1, 1, _G * _K), jnp.float32)
    ivec = jnp.zeros((1, 1, _G * _K), jnp.int32)
    bvec = jnp.zeros((1, 1, _G * _K), jnp.int32)
    for g in range(_G):
        b0 = bias_ref[0, 0, g]
        b1 = bias_ref[0, 0, g + _G]
        lp0 = lprobs_ref[0, g, :].reshape(1, V) + b0 - pen
        lp1 = lprobs_ref[0, g + _G, :].reshape(1, V) + b1 - pen
        toks = []
        for i in range(_K):
            m0 = jnp.max(lp0)
            m1 = jnp.max(lp1)
            m = jnp.maximum(m0, m1)
            any0 = m0 >= m1
            i0 = jnp.min(jnp.where(lp0 == m, iota, BIG))
            i1 = jnp.min(jnp.where(lp1 == m, iota, BIG))
            tok = jnp.where(any0, i0, i1).astype(jnp.int32)
            sub = jnp.where(any0, 0, 1).astype(jnp.int32)
            slot = oiota == (i * _G + g)
            svec = jnp.where(slot, m, svec)
            ivec = jnp.where(slot, tok, ivec)
            bvec = jnp.where(slot, sub * _G + g, bvec)
            hit = iota == tok
            lp0 = jnp.where(hit & any0, _NEG, lp0)
            lp1 = jnp.where(hit & jnp.logical_not(any0), _NEG, lp1)
            toks.append(tok)
        if g < _G - 1:
            for t in toks:
                pen = pen + _ALPHA * (iota == t).astype(jnp.float32)
    outs_ref[...] = svec
    outi_ref[...] = ivec
    outb_ref[...] = bvec


def kernel(step, lprobs, scores):
    bsz, beam, vocab = lprobs.shape
    bias = jax.lax.dynamic_slice_in_dim(scores, step - 1, 1, axis=2)
    bias = bias.reshape(bsz, 1, beam)
    out_shapes = [
        jax.ShapeDtypeStruct((bsz, 1, _G * _K), jnp.float32),
        jax.ShapeDtypeStruct((bsz, 1, _G * _K), jnp.int32),
        jax.ShapeDtypeStruct((bsz, 1, _G * _K), jnp.int32),
    ]
    in_specs = [
        pl.BlockSpec((1, beam, vocab), lambda b: (b, 0, 0)),
        pl.BlockSpec((1, 1, beam), lambda b: (b, 0, 0)),
    ]
    out_specs = [pl.BlockSpec((1, 1, _G * _K), lambda b: (b, 0, 0))] * 3
    sc, idx, bm = pl.pallas_call(
        _tc_body,
        grid=(bsz,),
        in_specs=in_specs,
        out_specs=out_specs,
        out_shape=out_shapes,
    )(lprobs, bias)
    shp = (bsz, _G * _K)
    return sc.reshape(shp), idx.reshape(shp), bm.reshape(shp)


# speedup vs baseline: 7.9517x; 5.5568x over previous
import jax
import jax.numpy as jnp
from jax.experimental import pallas as pl

_G = 4
_ALPHA = 0.5
_K = 4
_NEG = -3.0e38
_R = 8


def _tc_body(lprobs_ref, bias_ref, outs_ref, outi_ref, outb_ref):
    C = lprobs_ref.shape[-1]
    f = (
        jax.lax.broadcasted_iota(jnp.int32, (_R, C), 0) * C
        + jax.lax.broadcasted_iota(jnp.int32, (_R, C), 1)
    )
    pen = jnp.zeros((_R, C), jnp.float32)
    BIG = jnp.int32(2**30)
    oiota = jax.lax.broadcasted_iota(jnp.int32, (1, 1, _G * _K), 2)
    svec = jnp.zeros((1, 1, _G * _K), jnp.float32)
    ivec = jnp.zeros((1, 1, _G * _K), jnp.int32)
    bvec = jnp.zeros((1, 1, _G * _K), jnp.int32)
    for g in range(_G):
        b0 = bias_ref[0, 0, g]
        b1 = bias_ref[0, 0, g + _G]
        lp0 = lprobs_ref[0, g] + b0 - pen
        lp1 = lprobs_ref[0, g + _G] + b1 - pen
        toks = []
        for i in range(_K):
            m0 = jnp.max(lp0)
            m1 = jnp.max(lp1)
            m = jnp.maximum(m0, m1)
            any0 = m0 >= m1
            i0 = jnp.min(jnp.where(lp0 == m, f, BIG))
            i1 = jnp.min(jnp.where(lp1 == m, f, BIG))
            tok = jnp.where(any0, i0, i1).astype(jnp.int32)
            sub = jnp.where(any0, 0, 1).astype(jnp.int32)
            slot = oiota == (i * _G + g)
            svec = jnp.where(slot, m, svec)
            ivec = jnp.where(slot, tok, ivec)
            bvec = jnp.where(slot, sub * _G + g, bvec)
            hit = f == tok
            lp0 = jnp.where(hit & any0, _NEG, lp0)
            lp1 = jnp.where(hit & jnp.logical_not(any0), _NEG, lp1)
            toks.append(tok)
        if g < _G - 1:
            for t in toks:
                pen = pen + _ALPHA * (f == t).astype(jnp.float32)
    outs_ref[...] = svec
    outi_ref[...] = ivec
    outb_ref[...] = bvec


def kernel(step, lprobs, scores):
    bsz, beam, vocab = lprobs.shape
    C = vocab // _R
    lp4 = lprobs.reshape(bsz, beam, _R, C)
    bias = jax.lax.dynamic_slice_in_dim(scores, step - 1, 1, axis=2)
    bias = bias.reshape(bsz, 1, beam)
    out_shapes = [
        jax.ShapeDtypeStruct((bsz, 1, _G * _K), jnp.float32),
        jax.ShapeDtypeStruct((bsz, 1, _G * _K), jnp.int32),
        jax.ShapeDtypeStruct((bsz, 1, _G * _K), jnp.int32),
    ]
    in_specs = [
        pl.BlockSpec((1, beam, _R, C), lambda b: (b, 0, 0, 0)),
        pl.BlockSpec((1, 1, beam), lambda b: (b, 0, 0)),
    ]
    out_specs = [pl.BlockSpec((1, 1, _G * _K), lambda b: (b, 0, 0))] * 3
    sc, idx, bm = pl.pallas_call(
        _tc_body,
        grid=(bsz,),
        in_specs=in_specs,
        out_specs=out_specs,
        out_shape=out_shapes,
    )(lp4, bias)
    shp = (bsz, _G * _K)
    return sc.reshape(shp), idx.reshape(shp), bm.reshape(shp)


# speedup vs baseline: 7.9611x; 1.0012x over previous
import jax
import jax.numpy as jnp
from jax.experimental import pallas as pl
from jax.experimental.pallas import tpu as pltpu

_G = 4
_ALPHA = 0.5
_K = 4
_NEG = -3.0e38
_R = 8


def _tc_body(lprobs_ref, bias_ref, outs_ref, outi_ref, outb_ref):
    C = lprobs_ref.shape[-1]
    f = (
        jax.lax.broadcasted_iota(jnp.int32, (_R, C), 0) * C
        + jax.lax.broadcasted_iota(jnp.int32, (_R, C), 1)
    )
    pen = jnp.zeros((_R, C), jnp.float32)
    BIG = jnp.int32(2**30)
    oiota = jax.lax.broadcasted_iota(jnp.int32, (1, 1, _G * _K), 2)
    svec = jnp.zeros((1, 1, _G * _K), jnp.float32)
    ivec = jnp.zeros((1, 1, _G * _K), jnp.int32)
    bvec = jnp.zeros((1, 1, _G * _K), jnp.int32)
    for g in range(_G):
        b0 = bias_ref[0, 0, g]
        b1 = bias_ref[0, 0, g + _G]
        lp0 = lprobs_ref[0, g] + b0 - pen
        lp1 = lprobs_ref[0, g + _G] + b1 - pen
        toks = []
        for i in range(_K):
            m0 = jnp.max(lp0)
            m1 = jnp.max(lp1)
            m = jnp.maximum(m0, m1)
            any0 = m0 >= m1
            i0 = jnp.min(jnp.where(lp0 == m, f, BIG))
            i1 = jnp.min(jnp.where(lp1 == m, f, BIG))
            tok = jnp.where(any0, i0, i1).astype(jnp.int32)
            sub = jnp.where(any0, 0, 1).astype(jnp.int32)
            slot = oiota == (i * _G + g)
            svec = jnp.where(slot, m, svec)
            ivec = jnp.where(slot, tok, ivec)
            bvec = jnp.where(slot, sub * _G + g, bvec)
            hit = f == tok
            lp0 = jnp.where(hit & any0, _NEG, lp0)
            lp1 = jnp.where(hit & jnp.logical_not(any0), _NEG, lp1)
            toks.append(tok)
        if g < _G - 1:
            for t in toks:
                pen = pen + _ALPHA * (f == t).astype(jnp.float32)
    outs_ref[...] = svec
    outi_ref[...] = ivec
    outb_ref[...] = bvec


def kernel(step, lprobs, scores):
    bsz, beam, vocab = lprobs.shape
    C = vocab // _R
    lp4 = lprobs.reshape(bsz, beam, _R, C)
    bias = jax.lax.dynamic_slice_in_dim(scores, step - 1, 1, axis=2)
    bias = bias.reshape(bsz, 1, beam)
    out_shapes = [
        jax.ShapeDtypeStruct((bsz, 1, _G * _K), jnp.float32),
        jax.ShapeDtypeStruct((bsz, 1, _G * _K), jnp.int32),
        jax.ShapeDtypeStruct((bsz, 1, _G * _K), jnp.int32),
    ]
    in_specs = [
        pl.BlockSpec((1, beam, _R, C), lambda b: (b, 0, 0, 0)),
        pl.BlockSpec((1, 1, beam), lambda b: (b, 0, 0)),
    ]
    out_specs = [pl.BlockSpec((1, 1, _G * _K), lambda b: (b, 0, 0))] * 3
    sc, idx, bm = pl.pallas_call(
        _tc_body,
        grid=(bsz,),
        in_specs=in_specs,
        out_specs=out_specs,
        out_shape=out_shapes,
        compiler_params=pltpu.CompilerParams(
            dimension_semantics=("parallel",)
        ),
    )(lp4, bias)
    shp = (bsz, _G * _K)
    return sc.reshape(shp), idx.reshape(shp), bm.reshape(shp)
